# Initial kernel scaffold; baseline (speedup 1.0000x reference)
import jax
import jax.numpy as jnp
from jax.experimental import pallas as pl


def kernel(queries, keys):
    raise NotImplementedError("write your pallas kernel here")


# speedup vs baseline: 6.8534x; 6.8534x over previous
import functools

import jax
import jax.numpy as jnp
from jax.experimental import pallas as pl
from jax.experimental.pallas import tpu as pltpu
from jax.experimental.pallas import tpu_sc as plsc

Q = 4096
D = 128
NK = 100000
C = 128
KPAD = 102400
M = KPAD // C
KB = 1024
CPK = KB // C
NKB = KPAD // KB
TOPK = 16
CAND = TOPK * C
QB = 512
NQB = Q // QB
NEG_INF = float("-inf")
BIG_I32 = 2**30


def _p1_body(q_ref, k_ref, sims_ref, cm_ref, qn_ref):
    kstep = pl.program_id(0)

    @pl.when(kstep == 0)
    def _():
        q = q_ref[...]
        n = jnp.sqrt(jnp.sum(q * q, axis=1, keepdims=True))
        qn_ref[...] = q / jnp.maximum(n, 1e-10)

    k = k_ref[...]
    n = jnp.sqrt(jnp.sum(k * k, axis=1, keepdims=True))
    kn = k / jnp.maximum(n, 1e-10)
    s = jax.lax.dot_general(qn_ref[...], kn, (((1,), (1,)), ((), ())),
                            preferred_element_type=jnp.float32)
    col_ids = kstep * KB + jax.lax.broadcasted_iota(jnp.int32, (Q, KB), 1)
    s = jnp.where(col_ids < NK, s, NEG_INF)
    sims_ref[...] = s
    cm = [jnp.max(s[:, j * C:(j + 1) * C], axis=1, keepdims=True)
          for j in range(CPK)]
    cm_ref[0] = jnp.concatenate(cm, axis=1)


def _p2_body(cm_ref, gidx_ref, base_ref):
    b = pl.program_id(0)
    v = cm_ref[...]
    chunk_ids = jax.lax.broadcasted_iota(jnp.int32, (QB, M), 1)
    qrow = b * QB + jax.lax.broadcasted_iota(jnp.int32, (QB, 1), 0)
    gcols, bcols = [], []
    for _ in range(TOPK):
        m = jnp.max(v, axis=1, keepdims=True)
        sel = jnp.min(jnp.where(v == m, chunk_ids, BIG_I32), axis=1,
                      keepdims=True)
        gcols.append(qrow * M + sel)
        bcols.append(sel * C)
        v = jnp.where(chunk_ids == sel, NEG_INF, v)
    gidx_ref[0] = jnp.concatenate(gcols, axis=1)
    base_ref[0] = jnp.concatenate(bcols, axis=1)


def _p4_body(g_ref, base_ref, vals_ref, idx_ref):
    v = g_ref[...]
    bases = base_ref[0]
    cols = jax.lax.broadcasted_iota(jnp.int32, (QB, C), 1)
    gidx = jnp.concatenate(
        [bases[:, i:i + 1] + cols for i in range(TOPK)], axis=1)
    vcols, icols = [], []
    for _ in range(TOPK):
        m = jnp.max(v, axis=1, keepdims=True)
        sel = jnp.min(jnp.where(v == m, gidx, BIG_I32), axis=1, keepdims=True)
        vcols.append(m)
        icols.append(sel)
        v = jnp.where(gidx == sel, NEG_INF, v)
    vals_ref[0] = jnp.concatenate(vcols, axis=1)
    idx_ref[0] = jnp.concatenate(icols, axis=1)


def _sc_gather(sims_rows, flat_idx):
    n_idx = flat_idx.shape[0]
    window = 128
    mesh = plsc.VectorSubcoreMesh(core_axis_name="core",
                                  subcore_axis_name="subcore")

    @functools.partial(
        pl.kernel,
        out_type=jax.ShapeDtypeStruct((n_idx, C), jnp.float32),
        mesh=mesh)
    def kern(x_hbm, i_hbm, o_hbm):
        def body(i_vmem, o_vmem):
            pltpu.sync_copy(x_hbm.at[i_vmem.at[0]], o_vmem)

        pltpu.emit_pipeline(
            body,
            grid=(n_idx // window,),
            in_specs=[pl.BlockSpec((1, window), index_map=lambda i: (0, i))],
            out_specs=[pl.BlockSpec((window, C), index_map=lambda i: (i, 0))],
            core_axis_name=("core", "subcore"),
            dimension_semantics=(pltpu.PARALLEL,),
        )(i_hbm, o_hbm)

    return kern(sims_rows, flat_idx.reshape(1, n_idx))


def kernel(queries, keys):
    keys_p = jnp.concatenate(
        [keys, jnp.zeros((KPAD - NK, D), jnp.float32)], axis=0)

    sims, cm3 = pl.pallas_call(
        _p1_body,
        grid=(NKB,),
        in_specs=[
            pl.BlockSpec((Q, D), lambda k: (0, 0)),
            pl.BlockSpec((KB, D), lambda k: (k, 0)),
        ],
        out_specs=[
            pl.BlockSpec((Q, KB), lambda k: (0, k)),
            pl.BlockSpec((1, Q, CPK), lambda k: (k, 0, 0)),
        ],
        out_shape=[
            jax.ShapeDtypeStruct((Q, KPAD), jnp.float32),
            jax.ShapeDtypeStruct((NKB, Q, CPK), jnp.float32),
        ],
        scratch_shapes=[pltpu.VMEM((Q, D), jnp.float32)],
    )(queries, keys_p)

    cm = jnp.transpose(cm3, (1, 0, 2)).reshape(Q, M)

    gidx3, base3 = pl.pallas_call(
        _p2_body,
        grid=(NQB,),
        in_specs=[pl.BlockSpec((QB, M), lambda b: (b, 0))],
        out_specs=[
            pl.BlockSpec((1, QB, TOPK), lambda b: (b, 0, 0)),
            pl.BlockSpec((1, QB, TOPK), lambda b: (b, 0, 0)),
        ],
        out_shape=[
            jax.ShapeDtypeStruct((NQB, QB, TOPK), jnp.int32),
            jax.ShapeDtypeStruct((NQB, QB, TOPK), jnp.int32),
        ],
    )(cm)

    gathered = _sc_gather(sims.reshape(Q * M, C),
                          gidx3.reshape(Q * TOPK))

    vals3, idx3 = pl.pallas_call(
        _p4_body,
        grid=(NQB,),
        in_specs=[
            pl.BlockSpec((QB, CAND), lambda b: (b, 0)),
            pl.BlockSpec((1, QB, TOPK), lambda b: (b, 0, 0)),
        ],
        out_specs=[
            pl.BlockSpec((1, QB, TOPK), lambda b: (b, 0, 0)),
            pl.BlockSpec((1, QB, TOPK), lambda b: (b, 0, 0)),
        ],
        out_shape=[
            jax.ShapeDtypeStruct((NQB, QB, TOPK), jnp.float32),
            jax.ShapeDtypeStruct((NQB, QB, TOPK), jnp.int32),
        ],
    )(gathered.reshape(Q, CAND), base3)

    return vals3.reshape(Q, TOPK), idx3.reshape(Q, TOPK)
